# baseline (speedup 1.0000x reference)
import functools

import jax
import jax.numpy as jnp
from jax import lax
from jax.experimental import pallas as pl
from jax.experimental.pallas import tpu as pltpu

_C = 256
_A = 9
_NCLS = 80
_TOT = 8525
_UN = 25

_LEVEL_CFG = {
    80: (96, 4, 10, 2),
    40: (48, 5, 10, 5),
    20: (32, 10, 20, 5),
    10: (16, 10, 10, 5),
    5: (16, 5, 5, 5),
}
_ROW_OFF = {80: 0, 40: 6400, 20: 8000, 10: 8400, 5: 8500}


def _conv_chunk(src, r0, MB, Wpb, wtaps):
    G = src[pl.ds(r0, MB + 2), :, :].reshape((MB + 2) * Wpb, _C)
    z = jnp.zeros((1, _C), jnp.bfloat16)
    Sm = jnp.concatenate([z, G[:-1]], axis=0)
    Sp = jnp.concatenate([G[1:], z], axis=0)
    Dout = wtaps[0][0].shape[-1]
    acc = jnp.zeros((MB * Wpb, Dout), jnp.float32)
    for ky in range(3):
        base = ky * Wpb
        for kx, sb in ((0, Sm), (1, G), (2, Sp)):
            lhs = sb[base:base + MB * Wpb]
            acc = acc + jnp.dot(lhs, wtaps[ky][kx],
                                preferred_element_type=jnp.float32)
    return acc


def _chunked(n, do_one):
    if n <= 5:
        for ci in range(n):
            do_one(ci)
    else:
        assert n % 2 == 0

        def body(t, carry):
            do_one(2 * t)
            do_one(2 * t + 1)
            return carry

        lax.fori_loop(0, n // 2, body, 0)


def _head_kernel(*args, S, W, Wpb, MB, RB, MBo, Do):
    x_ref, tw_ref, tb_ref, ow_ref, ob_ref = args[:5]
    out_ref, xb, pb = args[-3:]
    j = pl.program_id(1)

    @pl.when(j == 0)
    def _tower():
        xb[0:1, :, :] = jnp.zeros((1, Wpb, _C), jnp.bfloat16)
        xb[S + 1:S + 2, :, :] = jnp.zeros((1, Wpb, _C), jnp.bfloat16)
        xb[:, W:Wpb, :] = jnp.zeros((S + 2, Wpb - W, _C), jnp.bfloat16)
        pb[0:1, :, :] = jnp.zeros((1, Wpb, _C), jnp.bfloat16)
        pb[S + 1:S + 2, :, :] = jnp.zeros((1, Wpb, _C), jnp.bfloat16)
        xb[1:S + 1, 0:W, :] = x_ref[0]
        for layer in range(4):
            src, dst = (xb, pb) if layer % 2 == 0 else (pb, xb)
            wks = [[tw_ref[layer, ky, kx] for kx in range(3)]
                   for ky in range(3)]
            bias = tb_ref[layer]

            def chunk(ci, src=src, dst=dst, wks=wks, bias=bias):
                r0 = ci * MB
                acc = _conv_chunk(src, r0, MB, Wpb, wks)
                y = jnp.maximum(acc + bias, 0.0).astype(jnp.bfloat16)
                y = y.reshape(MB, Wpb, _C)
                col = lax.broadcasted_iota(jnp.int32, (MB, Wpb, _C), 1)
                y = jnp.where(col < W, y, jnp.bfloat16(0))
                dst[pl.ds(r0 + 1, MB), :, :] = y

            _chunked(S // MB, chunk)

    ows = [[ow_ref[ky, kx] for kx in range(3)] for ky in range(3)]
    ob = ob_ref[...]

    for ci in range(RB // MBo):
        r0 = j * RB + ci * MBo
        acc = _conv_chunk(xb, r0, MBo, Wpb, ows)
        acc3 = (acc + ob).reshape(MBo, Wpb, Do)
        for m in range(MBo):
            p = (ci * MBo + m) * W
            c = 0
            while c < W:
                u, q = divmod(p + c, _UN)
                take = min(_UN - q, W - c)
                out_ref[0, u, q:q + take, :] = acc3[m, c:c + take, :]
                c += take


def _run_head(x, tw, tb, ow, obias, *, S, W, Wpb, MB, RB, MBo, Do, name,
              big=None, interpret=False):
    N = x.shape[0]
    NB = S // RB
    U = RB * W // _UN
    off_blocks = _ROW_OFF[S] // _UN // U
    assert _ROW_OFF[S] // _UN % U == 0 and RB * W % _UN == 0
    kern = functools.partial(_head_kernel, S=S, W=W, Wpb=Wpb, MB=MB, RB=RB,
                             MBo=MBo, Do=Do)
    in_specs = [
        pl.BlockSpec((1, S, W, _C), lambda n, j: (n, 0, 0, 0)),
        pl.BlockSpec((4, 3, 3, _C, _C), lambda n, j: (0, 0, 0, 0, 0)),
        pl.BlockSpec((4, 1, _C), lambda n, j: (0, 0, 0)),
        pl.BlockSpec((3, 3, _C, Do), lambda n, j: (0, 0, 0, 0)),
        pl.BlockSpec((1, Do), lambda n, j: (0, 0)),
    ]
    inputs = [x, tw, tb, ow, obias]
    aliases = {}
    out_specs = pl.BlockSpec(
        (1, U, _UN, Do), lambda n, j, off=off_blocks: (n, off + j, 0, 0))
    out_shape = jax.ShapeDtypeStruct((N, _TOT // _UN, _UN, Do), jnp.float32)
    if big is not None:
        in_specs.append(pl.BlockSpec(memory_space=pl.ANY))
        inputs.append(big)
        aliases = {5: 0}
    return pl.pallas_call(
        kern,
        grid=(N, NB),
        in_specs=in_specs,
        out_specs=out_specs,
        out_shape=out_shape,
        input_output_aliases=aliases,
        scratch_shapes=[
            pltpu.VMEM((S + 2, Wpb, _C), jnp.bfloat16),
            pltpu.VMEM((S + 2, Wpb, _C), jnp.bfloat16),
        ],
        compiler_params=pltpu.CompilerParams(
            dimension_semantics=("parallel", "arbitrary"),
            vmem_limit_bytes=100 * 1024 * 1024,
        ),
        name=name,
        interpret=interpret,
    )(*inputs)


def kernel(x0, x1, x2, x3, x4,
           cls_conv_w, cls_conv_b, cls_out_w, cls_out_b,
           reg_conv_w, reg_conv_b, reg_out_w, reg_out_b):
    feats = [x0, x1, x2, x3, x4]
    N = x0.shape[0]

    def prep_head(conv_w, conv_b, out_w, out_b):
        tw = jnp.transpose(conv_w, (0, 3, 4, 2, 1)).astype(jnp.bfloat16)
        tb = conv_b.astype(jnp.float32).reshape(4, 1, _C)
        ow = jnp.transpose(out_w, (2, 3, 1, 0)).astype(jnp.bfloat16)
        obias = out_b.astype(jnp.float32).reshape(1, -1)
        return tw, tb, ow, obias

    heads = {
        "cls": (prep_head(cls_conv_w, cls_conv_b, cls_out_w, cls_out_b),
                _A * _NCLS),
        "reg": (prep_head(reg_conv_w, reg_conv_b, reg_out_w, reg_out_b),
                _A * 4),
    }
    xhs = {f.shape[2]: jnp.transpose(f, (0, 2, 3, 1)).astype(jnp.bfloat16)
           for f in feats}

    outs = {}
    for hname, (hp, Do) in heads.items():
        big = None
        for S in (80, 40, 20, 10, 5):
            Wpb, MB, RB, MBo = _LEVEL_CFG[S]
            big = _run_head(xhs[S], *hp, S=S, W=S, Wpb=Wpb, MB=MB, RB=RB,
                            MBo=MBo, Do=Do, name=f"retina_{hname}_{S}",
                            big=big)
        outs[hname] = big.reshape(N, _TOT * _A, Do // _A)
    return outs["cls"], outs["reg"]
